# baseline (speedup 1.0000x reference)
import jax
import jax.numpy as jnp
from jax import lax
from jax.experimental import pallas as pl
from jax.experimental.pallas import tpu as pltpu
from jax.experimental.pallas import tpu_sc as plsc

_NC = 2
_NS = 16
_NW = _NC * _NS
_NPAIR = _NW // 2
_C = 64
_K = 100
_PER_PAIR = _C * _K
_NPAD = _NPAIR * _PER_PAIR
_H = 256
_HH = _H // 2
_F = 9
_ROWS = 900


def _sc_body(gidx_hbm, emb_hbm, out_hbm, table_v, idx_v, out_v, sem):
    c = lax.axis_index("c")
    s = lax.axis_index("s")
    wid = s * _NC + c
    half = wid % 2
    pair = wid // 2

    pltpu.sync_copy(emb_hbm.at[half], table_v)

    lanes = lax.iota(jnp.int32, 16)

    def chunk_body(k, carry):
        pltpu.sync_copy(gidx_hbm.at[pair, k], idx_v)

        for g in range(_C // 16):
            base = [idx_v[i, pl.ds(g * 16, 16)] for i in range(_F)]
            nodes16 = lanes + (g * 16)

            def col_body(col, carry2):
                colperm = (lanes + col) & (_HH - 1)
                acc = plsc.load_gather(table_v, [base[0] + colperm])
                for i in range(1, _F):
                    acc = acc + plsc.load_gather(table_v, [base[i] + colperm])
                plsc.store_scatter(out_v, [nodes16, colperm], acc)
                return carry2

            lax.fori_loop(0, _HH, col_body, 0, unroll=8)

        pltpu.sync_copy(
            out_v,
            out_hbm.at[pl.ds(pair * _PER_PAIR + k * _C, _C),
                       pl.ds(half * _HH, _HH)],
        )
        return carry

    lax.fori_loop(0, _K, chunk_body, 0, unroll=False)


def kernel(x, emb):
    n, f = x.shape
    _, v, h = emb.shape
    gidx = (x + v * jnp.arange(f, dtype=jnp.int32)[None, :]) * _HH
    gidx = jnp.zeros((_NPAD, f), jnp.int32).at[:n].set(gidx)
    gidx4 = gidx.reshape(_NPAIR, _K, _C, f).transpose(0, 1, 3, 2)
    emb_flat = emb.reshape(f * v, h)
    emb_halves = jnp.stack(
        [emb_flat[:, :_HH].reshape(-1), emb_flat[:, _HH:].reshape(-1)]
    )

    mesh = plsc.VectorSubcoreMesh(
        core_axis_name="c", subcore_axis_name="s",
        num_cores=_NC, num_subcores=_NS,
    )
    run = pl.kernel(
        _sc_body,
        out_type=jax.ShapeDtypeStruct((_NPAD, h), jnp.float32),
        mesh=mesh,
        scratch_types=[
            pltpu.VMEM((_ROWS * _HH,), jnp.float32),
            pltpu.VMEM((_F, _C), jnp.int32),
            pltpu.VMEM((_C, _HH), jnp.float32),
            pltpu.SemaphoreType.DMA,
        ],
        compiler_params=pltpu.CompilerParams(needs_layout_passes=False),
    )
    out = run(gidx4, emb_halves)
    return out[:n]


# speedup vs baseline: 1.7795x; 1.7795x over previous
import jax
import jax.numpy as jnp
from jax import lax
from jax.experimental import pallas as pl
from jax.experimental.pallas import tpu as pltpu
from jax.experimental.pallas import tpu_sc as plsc

_NC = 2
_NS = 16
_NW = _NC * _NS
_NPAIR = _NW // 2
_C = 64
_H = 256
_HH = _H // 2
_F = 9
_ROWS = 900

_K = 50
_PER_PAIR = _C * _K
_SC_NODES = _NPAIR * _PER_PAIR

_B = 2048


def _sc_body(gidx_hbm, emb_hbm, out_hbm, table_v, idx_v, out_v, sem):
    c = lax.axis_index("c")
    s = lax.axis_index("s")
    wid = s * _NC + c
    half = wid % 2
    pair = wid // 2

    pltpu.sync_copy(emb_hbm.at[half], table_v)

    lanes = lax.iota(jnp.int32, 16)

    def chunk_body(k, carry):
        pltpu.sync_copy(gidx_hbm.at[pair, k], idx_v)

        for g in range(_C // 16):
            base = [idx_v[i, pl.ds(g * 16, 16)] for i in range(_F)]
            nodes16 = lanes + (g * 16)

            def col_body(col, carry2):
                colperm = (lanes + col) & (_HH - 1)
                acc = plsc.load_gather(table_v, [base[0] + colperm])
                for i in range(1, _F):
                    acc = acc + plsc.load_gather(table_v, [base[i] + colperm])
                plsc.store_scatter(out_v, [nodes16, colperm], acc)
                return carry2

            lax.fori_loop(0, _HH, col_body, 0, unroll=8)

        pltpu.sync_copy(
            out_v,
            out_hbm.at[pl.ds(pair * _PER_PAIR + k * _C, _C),
                       pl.ds(half * _HH, _HH)],
        )
        return carry

    lax.fori_loop(0, _K, chunk_body, 0, unroll=False)


def _sc_part(x_sc, emb_flat):
    gidx = (x_sc + 100 * jnp.arange(_F, dtype=jnp.int32)[None, :]) * _HH
    gidx4 = gidx.reshape(_NPAIR, _K, _C, _F).transpose(0, 1, 3, 2)
    emb_halves = jnp.stack(
        [emb_flat[:, :_HH].reshape(-1), emb_flat[:, _HH:].reshape(-1)]
    )
    mesh = plsc.VectorSubcoreMesh(
        core_axis_name="c", subcore_axis_name="s",
        num_cores=_NC, num_subcores=_NS,
    )
    run = pl.kernel(
        _sc_body,
        out_type=jax.ShapeDtypeStruct((_SC_NODES, _H), jnp.float32),
        mesh=mesh,
        scratch_types=[
            pltpu.VMEM((_ROWS * _HH,), jnp.float32),
            pltpu.VMEM((_F, _C), jnp.int32),
            pltpu.VMEM((_C, _HH), jnp.float32),
            pltpu.SemaphoreType.DMA,
        ],
        compiler_params=pltpu.CompilerParams(needs_layout_passes=False),
    )
    return run(gidx4, emb_halves)


def _tc_body(xt_ref, emb_ref, out_ref):
    xt = xt_ref[...]
    f, b = xt.shape
    v = emb_ref.shape[0] // f
    iota = lax.broadcasted_iota(jnp.int32, (v, b), 0)
    parts = [(xt[i : i + 1] == iota).astype(jnp.float32) for i in range(f)]
    oh_t = jnp.concatenate(parts, axis=0)
    out_ref[...] = lax.dot_general(
        oh_t,
        emb_ref[...],
        (((0,), (0,)), ((), ())),
        preferred_element_type=jnp.float32,
    )


def _tc_part(x_tc, emb_flat):
    m, f = x_tc.shape
    grid = -(-m // _B)
    m_pad = grid * _B
    xt = jnp.zeros((f, m_pad), jnp.int32).at[:, :m].set(x_tc.T)
    out = pl.pallas_call(
        _tc_body,
        grid=(grid,),
        in_specs=[
            pl.BlockSpec((f, _B), lambda i: (0, i)),
            pl.BlockSpec((_F * 100, _H), lambda i: (0, 0)),
        ],
        out_specs=pl.BlockSpec((_B, _H), lambda i: (i, 0)),
        out_shape=jax.ShapeDtypeStruct((m_pad, _H), jnp.float32),
    )(xt, emb_flat)
    return out[:m]


def kernel(x, emb):
    n, f = x.shape
    _, v, h = emb.shape
    emb_flat = emb.reshape(f * v, h)
    m = n - _SC_NODES
    out_tc = _tc_part(x[:m], emb_flat)
    out_sc = _sc_part(x[m:], emb_flat)
    return jnp.concatenate([out_tc, out_sc], axis=0)
